# Initial kernel scaffold; baseline (speedup 1.0000x reference)
import jax
import jax.numpy as jnp
from jax.experimental import pallas as pl


def kernel(input_, W):
    raise NotImplementedError("write your pallas kernel here")


# speedup vs baseline: 1.4139x; 1.4139x over previous
import functools

import jax
import jax.numpy as jnp
from jax import lax
from jax.experimental import pallas as pl
from jax.experimental.pallas import tpu as pltpu
from jax.experimental.pallas import tpu_sc as plsc

NUM_POSITIONS = 2048
D = 2048
B = 4 * 2048

NC, NS = 2, 16
NW = NC * NS
B_PER_W = B // NW
CHUNK = 16
NCHUNK = B_PER_W // CHUNK


def _gather_rows(table, idx_flat):
    mesh = plsc.VectorSubcoreMesh(core_axis_name="c", subcore_axis_name="s")

    @functools.partial(
        pl.kernel,
        out_type=jax.ShapeDtypeStruct((B, D), jnp.float32),
        mesh=mesh,
        scratch_types=[
            pltpu.VMEM((B_PER_W,), jnp.int32),
            pltpu.VMEM((CHUNK, D), jnp.float32),
            pltpu.SemaphoreType.DMA,
        ],
    )
    def k(table_hbm, idx_hbm, out_hbm, idx_v, rows_v, sem):
        wid = lax.axis_index("s") * NC + lax.axis_index("c")
        base = wid * B_PER_W
        pltpu.sync_copy(idx_hbm.at[pl.ds(base, B_PER_W)], idx_v)
        for c in range(NCHUNK):
            pltpu.async_copy(
                table_hbm.at[idx_v.at[pl.ds(c * CHUNK, CHUNK)]], rows_v, sem
            ).wait()
            pltpu.sync_copy(rows_v, out_hbm.at[pl.ds(base + c * CHUNK, CHUNK)])

    return k(table, idx_flat)


def kernel(input_, W):
    idx_flat = input_.reshape(B).astype(jnp.int32)
    out = _gather_rows(W, idx_flat)
    return out.reshape(input_.shape[0], input_.shape[1], NUM_POSITIONS)


# speedup vs baseline: 1.5619x; 1.1047x over previous
import functools

import jax
import jax.numpy as jnp
from jax import lax
from jax.experimental import pallas as pl
from jax.experimental.pallas import tpu as pltpu
from jax.experimental.pallas import tpu_sc as plsc

NUM_POSITIONS = 2048
D = 2048
B = 4 * 2048

NC, NS = 2, 16
NW = NC * NS
B_PER_W = B // NW
CHUNK = 16
NCHUNK = B_PER_W // CHUNK
NBUF = 2


def _gather_rows(table, idx_flat):
    mesh = plsc.VectorSubcoreMesh(core_axis_name="c", subcore_axis_name="s")

    @functools.partial(
        pl.kernel,
        out_type=jax.ShapeDtypeStruct((B, D), jnp.float32),
        mesh=mesh,
        scratch_types=[
            pltpu.VMEM((B_PER_W,), jnp.int32),
            pltpu.VMEM((CHUNK, D), jnp.float32),
            pltpu.VMEM((CHUNK, D), jnp.float32),
            pltpu.SemaphoreType.DMA,
            pltpu.SemaphoreType.DMA,
            pltpu.SemaphoreType.DMA,
            pltpu.SemaphoreType.DMA,
        ],
    )
    def k(table_hbm, idx_hbm, out_hbm, idx_v, rows0, rows1, g0, g1, o0, o1):
        wid = lax.axis_index("s") * NC + lax.axis_index("c")
        base = wid * B_PER_W
        pltpu.sync_copy(idx_hbm.at[pl.ds(base, B_PER_W)], idx_v)

        bufs = (rows0, rows1)
        gsem = (g0, g1)
        osem = (o0, o1)

        def issue_gather(c):
            b = c % NBUF
            return pltpu.async_copy(
                table_hbm.at[idx_v.at[pl.ds(c * CHUNK, CHUNK)]],
                bufs[b],
                gsem[b],
            )

        out_cp = [None] * NBUF
        g = issue_gather(0)
        for c in range(NCHUNK):
            b = c % NBUF
            g.wait()
            if c + 1 < NCHUNK:
                nb = (c + 1) % NBUF
                if out_cp[nb] is not None:
                    out_cp[nb].wait()
                g = issue_gather(c + 1)
            out_cp[b] = pltpu.async_copy(
                bufs[b],
                out_hbm.at[pl.ds(base + c * CHUNK, CHUNK)],
                osem[b],
            )
        for cp in out_cp:
            if cp is not None:
                cp.wait()

    return k(table, idx_flat)


def kernel(input_, W):
    idx_flat = input_.reshape(B).astype(jnp.int32)
    out = _gather_rows(W, idx_flat)
    return out.reshape(input_.shape[0], input_.shape[1], NUM_POSITIONS)


# speedup vs baseline: 1.6420x; 1.0513x over previous
import functools

import jax
import jax.numpy as jnp
from jax import lax
from jax.experimental import pallas as pl
from jax.experimental.pallas import tpu as pltpu
from jax.experimental.pallas import tpu_sc as plsc

NUM_POSITIONS = 2048
D = 2048
B = 4 * 2048

NC, NS = 2, 16
NW = NC * NS
B_PER_W = B // NW
CHUNK = 16
NCHUNK = B_PER_W // CHUNK
NBUF = 3


def _gather_rows(table, idx_flat):
    mesh = plsc.VectorSubcoreMesh(core_axis_name="c", subcore_axis_name="s")

    @functools.partial(
        pl.kernel,
        out_type=jax.ShapeDtypeStruct((B, D), jnp.float32),
        mesh=mesh,
        scratch_types=[
            pltpu.VMEM((B_PER_W,), jnp.int32),
            pltpu.VMEM((CHUNK, D), jnp.float32),
            pltpu.VMEM((CHUNK, D), jnp.float32),
            pltpu.VMEM((CHUNK, D), jnp.float32),
            pltpu.SemaphoreType.DMA,
            pltpu.SemaphoreType.DMA,
            pltpu.SemaphoreType.DMA,
            pltpu.SemaphoreType.DMA,
            pltpu.SemaphoreType.DMA,
            pltpu.SemaphoreType.DMA,
        ],
    )
    def k(table_hbm, idx_hbm, out_hbm, idx_v, rows0, rows1, rows2,
          g0, g1, g2, o0, o1, o2):
        wid = lax.axis_index("s") * NC + lax.axis_index("c")
        base = wid * B_PER_W
        pltpu.sync_copy(idx_hbm.at[pl.ds(base, B_PER_W)], idx_v)

        bufs = (rows0, rows1, rows2)
        gsem = (g0, g1, g2)
        osem = (o0, o1, o2)

        def issue_gather(c):
            b = c % NBUF
            return pltpu.async_copy(
                table_hbm.at[idx_v.at[pl.ds(c * CHUNK, CHUNK)]],
                bufs[b],
                gsem[b],
            )

        gather_cp = [None] * NBUF
        out_cp = [None] * NBUF
        for c in range(min(NBUF, NCHUNK)):
            gather_cp[c] = issue_gather(c)
        for c in range(NCHUNK):
            b = c % NBUF
            gather_cp[b].wait()
            out_cp[b] = pltpu.async_copy(
                bufs[b],
                out_hbm.at[pl.ds(base + c * CHUNK, CHUNK)],
                osem[b],
            )
            n = c + NBUF
            if n < NCHUNK:
                out_cp[b].wait()
                gather_cp[b] = issue_gather(n)
                out_cp[b] = None
        for cp in out_cp:
            if cp is not None:
                cp.wait()

    return k(table, idx_flat)


def kernel(input_, W):
    idx_flat = input_.reshape(B).astype(jnp.int32)
    out = _gather_rows(W, idx_flat)
    return out.reshape(input_.shape[0], input_.shape[1], NUM_POSITIONS)
